# baseline (speedup 1.0000x reference)
import functools

import jax
import jax.numpy as jnp
import numpy as np
from jax import lax
from jax.experimental import pallas as pl
from jax.experimental.pallas import tpu as pltpu
from jax.experimental.pallas import tpu_sc as plsc

N = 10000
E = 320000
DFEAT = 128
DG = 32
DE = 16
D = 128
H = 8
DH = D // H
L = 3

NP1 = N + 1
BLK = 128
NPAD = ((NP1 + BLK - 1) // BLK) * BLK

NC = 2
NS = 16
NW = NC * NS
NTAB = 10112
RPT = NTAB // NS
EPW = E // NW
K = 80
DNR = NTAB // 16


def _ln(x, s, b):
    m = x.mean(-1, keepdims=True)
    v = x.var(-1, keepdims=True)
    return (x - m) / jnp.sqrt(v + 1e-6) * s + b


def _encoder_body(nf_ref, w1_ref, b1_ref, lns_ref, lnb_ref, w2_ref, b2_ref,
                  out_ref):
    x = nf_ref[...]
    h = jnp.dot(x, w1_ref[...], preferred_element_type=jnp.float32) + b1_ref[...]
    m = h.mean(-1, keepdims=True)
    v = jnp.mean((h - m) * (h - m), axis=-1, keepdims=True)
    h = (h - m) * jax.lax.rsqrt(v + 1e-6) * lns_ref[...] + lnb_ref[...]
    h = jnp.maximum(h, 0.0)
    out_ref[...] = jnp.dot(h, w2_ref[...], preferred_element_type=jnp.float32) + b2_ref[...]


def _encoder(nf_pad, p):
    grid = NPAD // BLK
    return pl.pallas_call(
        _encoder_body,
        grid=(grid,),
        in_specs=[
            pl.BlockSpec((BLK, DFEAT + DG), lambda i: (i, 0)),
            pl.BlockSpec((DFEAT + DG, D), lambda i: (0, 0)),
            pl.BlockSpec((D,), lambda i: (0,)),
            pl.BlockSpec((D,), lambda i: (0,)),
            pl.BlockSpec((D,), lambda i: (0,)),
            pl.BlockSpec((D, D), lambda i: (0, 0)),
            pl.BlockSpec((D,), lambda i: (0,)),
        ],
        out_specs=pl.BlockSpec((BLK, D), lambda i: (i, 0)),
        out_shape=jax.ShapeDtypeStruct((NPAD, D), jnp.float32),
    )(nf_pad, p['enc_W1'], p['enc_b1'], p['enc_ln_s'], p['enc_ln_b'],
      p['enc_W2'], p['enc_b2'])


def _sc_edge_body(hs_tab, hr_tab, he_hbm, snd, rcv,
                  out_num, out_den,
                  acc, den_acc, att_v, idx_s, idx_r, idx_r2,
                  hs_rows, hr_rows, he_rows, exbuf, staged_den, zbuf,
                  sem_s, sem_r, sem_e):
    cid = lax.axis_index("c")
    sid = lax.axis_index("s")
    wid = cid * NS + sid
    zero16 = jnp.zeros((16,), jnp.float32)
    iota16 = lax.iota(jnp.int32, 16)

    def zrow(r, carry):
        for c in range(8):
            zbuf[r, pl.ds(c * 16, 16)] = zero16
        return carry
    lax.fori_loop(0, 8, zrow, 0)

    def zacc(t, carry):
        pltpu.sync_copy(zbuf, acc.at[pl.ds(sid * RPT + t * 8, 8)])
        return carry
    lax.fori_loop(0, RPT // 8, zacc, 0)
    dstart = jnp.minimum(sid * 40, DNR - 40)
    for t in range(5):
        pltpu.sync_copy(zbuf, den_acc.at[pl.ds(dstart + t * 8, 8)])

    def zsd(r, carry):
        for c in range(8):
            staged_den[r, pl.ds(c * 16, 16)] = zero16
        return carry
    lax.fori_loop(0, K, zsd, 0)
    pltpu.sync_copy(hs_tab.at[pl.ds(NTAB, 1)], att_v)
    plsc.subcore_barrier()

    ebase = wid * EPW

    def batch(b, carry):
        off = ebase + b * K
        pltpu.sync_copy(snd.at[pl.ds(off, K)], idx_s)
        pltpu.sync_copy(rcv.at[pl.ds(off, K)], idx_r)
        cs = pltpu.async_copy(hs_tab.at[idx_s], hs_rows, sem_s)
        cr = pltpu.async_copy(hr_tab.at[idx_r], hr_rows, sem_r)
        ce = pltpu.async_copy(he_hbm.at[pl.ds(off, K)], he_rows, sem_e)
        cs.wait()
        cr.wait()
        ce.wait()

        def p1(k, c1):
            for j in range(H):
                sl = pl.ds(j * 16, 16)
                mv = hs_rows[k, sl] + hr_rows[k, sl] + he_rows[k, sl]
                he_rows[k, sl] = jnp.maximum(mv, mv * 0.2)
            return c1
        lax.fori_loop(0, K, p1, 0, unroll=4)

        atts = [att_v[0, pl.ds(h * 16, 16)][dd]
                for h in range(H) for dd in range(16)]

        def p2(g, c2):
            rows = g * 16 + iota16
            rvec = idx_r[pl.ds(g * 16, 16)]
            idx_r2[pl.ds(g * 16, 16)] = lax.shift_right_logical(rvec, 4)
            posv = (rvec & 15) * 8
            for h in range(H):
                lg = jnp.zeros((16,), jnp.float32)
                for dd in range(16):
                    col = plsc.load_gather(
                        he_rows, [rows, jnp.full((16,), h * 16 + dd, jnp.int32)])
                    lg = lg + col * atts[h * 16 + dd]
                exh = jnp.exp(lg)
                plsc.store_scatter(exbuf, [g * 256 + iota16 * 16 + h], exh)
                plsc.store_scatter(staged_den, [rows, posv + h], exh)
            return c2
        lax.fori_loop(0, K // 16, p2, 0)

        def p3(k, c3):
            exrow = exbuf[pl.ds(k * 16, 16)]
            for j in range(H):
                sl = pl.ds(j * 16, 16)
                hr_rows[k, sl] = hs_rows[k, sl] * exrow[j]
            return c3
        lax.fori_loop(0, K, p3, 0, unroll=4)

        a1 = pltpu.async_copy(hr_rows, acc.at[idx_r], sem_s, add=True)
        a2 = pltpu.async_copy(staged_den, den_acc.at[idx_r2], sem_r, add=True)
        a2.wait()

        def pc(g, c4):
            rows = g * 16 + iota16
            posv = (idx_r[pl.ds(g * 16, 16)] & 15) * 8
            for h in range(H):
                plsc.store_scatter(staged_den, [rows, posv + h], zero16)
            return c4
        lax.fori_loop(0, K // 16, pc, 0)
        a1.wait()
        return carry
    lax.fori_loop(0, EPW // K, batch, 0)

    plsc.subcore_barrier()
    pltpu.sync_copy(acc.at[pl.ds(sid * RPT, RPT)],
                    out_num.at[cid, pl.ds(sid * RPT, RPT)])
    dstart2 = jnp.minimum(sid * 40, DNR - 40)
    pltpu.sync_copy(den_acc.at[pl.ds(dstart2, 40)],
                    out_den.at[cid, pl.ds(dstart2, 40)])


_sc_edges_built = None


def _build_sc_edges():
    return pl.kernel(
        _sc_edge_body,
        out_type=(jax.ShapeDtypeStruct((NC, NTAB, D), jnp.float32),
                  jax.ShapeDtypeStruct((NC, DNR, D), jnp.float32)),
        mesh=plsc.VectorSubcoreMesh(core_axis_name="c", subcore_axis_name="s",
                                    num_cores=NC, num_subcores=NS),
        scratch_types=[
            pltpu.VMEM_SHARED((NTAB, D), jnp.float32),
            pltpu.VMEM_SHARED((DNR, D), jnp.float32),
            pltpu.VMEM((1, D), jnp.float32),
            pltpu.VMEM((K,), jnp.int32),
            pltpu.VMEM((K,), jnp.int32),
            pltpu.VMEM((K,), jnp.int32),
            pltpu.VMEM((K, D), jnp.float32),
            pltpu.VMEM((K, D), jnp.float32),
            pltpu.VMEM((K, D), jnp.float32),
            pltpu.VMEM((K * 16,), jnp.float32),
            pltpu.VMEM((K, D), jnp.float32),
            pltpu.VMEM((8, D), jnp.float32),
            pltpu.SemaphoreType.DMA,
            pltpu.SemaphoreType.DMA,
            pltpu.SemaphoreType.DMA,
        ],
        compiler_params=pltpu.CompilerParams(needs_layout_passes=False),
        name="sc_gatv2_edges",
    )


def _sc_edges(*args):
    global _sc_edges_built
    if _sc_edges_built is None:
        _sc_edges_built = _build_sc_edges()
    return _sc_edges_built(*args)


_SEG = np.kron(np.eye(H, dtype=np.float32), np.ones((DH, 1), np.float32))


_P = np.kron(np.eye(H, dtype=np.float32), np.ones((DH, DH), np.float32))


def _attn_body(nodes_ref, nm_ref, wk_ref, bk_ref, wv_ref, bv_ref,
               klns_ref, klnb_ref, q_ref, pmat_ref,
               out_ref, macc_ref, sacc_ref, vacc_ref):
    i = pl.program_id(0)

    @pl.when(i == 0)
    def _init():
        macc_ref[...] = jnp.full((1, D), -1e30, jnp.float32)
        sacc_ref[...] = jnp.zeros((1, D), jnp.float32)
        vacc_ref[...] = jnp.zeros((1, D), jnp.float32)

    x = nodes_ref[...]
    pmat = pmat_ref[...]
    k = jnp.dot(x, wk_ref[...], preferred_element_type=jnp.float32) + bk_ref[...]
    mfull = jnp.dot(k, pmat, preferred_element_type=jnp.float32) * (1.0 / DH)
    kc = k - mfull
    vfull = jnp.dot(kc * kc, pmat, preferred_element_type=jnp.float32) * (1.0 / DH)
    kln = kc * jax.lax.rsqrt(vfull + 1e-6) * klns_ref[...] + klnb_ref[...]
    sfull = jnp.dot(kln * q_ref[...], pmat,
                    preferred_element_type=jnp.float32) * (1.0 / np.sqrt(DH))
    sfull = jnp.where(nm_ref[...] > 0.0, sfull, -1e9)
    bm = jnp.max(sfull, axis=0, keepdims=True)
    m_old = macc_ref[...]
    m_new = jnp.maximum(m_old, bm)
    corr = jnp.exp(m_old - m_new)
    w = jnp.exp(sfull - m_new)
    v = jnp.dot(x, wv_ref[...], preferred_element_type=jnp.float32) + bv_ref[...]
    macc_ref[...] = m_new
    sacc_ref[...] = sacc_ref[...] * corr + jnp.sum(w, axis=0, keepdims=True)
    vacc_ref[...] = vacc_ref[...] * corr + jnp.sum(w * v, axis=0, keepdims=True)

    @pl.when(i == pl.num_programs(0) - 1)
    def _fin():
        out_ref[...] = vacc_ref[...] / sacc_ref[...]


def _attention_pallas(g, nodes_pad, nm_pad, ap):
    q = (g @ ap['Wq'] + ap['bq']).reshape(1, H, DH)
    q = _ln(q, ap['qln_s'], ap['qln_b']).reshape(1, D)
    klns = jnp.tile(ap['kln_s'], H).reshape(1, D)
    klnb = jnp.tile(ap['kln_b'], H).reshape(1, D)
    out = pl.pallas_call(
        _attn_body,
        grid=(NPAD // BLK,),
        in_specs=[
            pl.BlockSpec((BLK, D), lambda i: (i, 0)),
            pl.BlockSpec((BLK, D), lambda i: (i, 0)),
            pl.BlockSpec((D, D), lambda i: (0, 0)),
            pl.BlockSpec((D,), lambda i: (0,)),
            pl.BlockSpec((D, D), lambda i: (0, 0)),
            pl.BlockSpec((D,), lambda i: (0,)),
            pl.BlockSpec((1, D), lambda i: (0, 0)),
            pl.BlockSpec((1, D), lambda i: (0, 0)),
            pl.BlockSpec((1, D), lambda i: (0, 0)),
            pl.BlockSpec((D, D), lambda i: (0, 0)),
        ],
        out_specs=pl.BlockSpec((1, D), lambda i: (0, 0)),
        out_shape=jax.ShapeDtypeStruct((1, D), jnp.float32),
        scratch_shapes=[
            pltpu.VMEM((1, D), jnp.float32),
            pltpu.VMEM((1, D), jnp.float32),
            pltpu.VMEM((1, D), jnp.float32),
        ],
    )(nodes_pad, jnp.repeat(nm_pad[:, None], D, axis=1),
      ap['Wk'], ap['bk'], ap['Wv'], ap['bv'],
      klns, klnb, q, jnp.asarray(_P))
    return g + out @ ap['Wo'] + ap['bo']


def _mm_ln_body(x_ref, w_ref, b_ref, out_ref, *, act):
    h = jnp.dot(x_ref[...], w_ref[...], preferred_element_type=jnp.float32) + b_ref[...]
    if act == 'relu':
        h = jnp.maximum(h, 0.0)
    out_ref[...] = h


def _mm(x_pad, w, b, act='none'):
    return pl.pallas_call(
        functools.partial(_mm_ln_body, act=act),
        grid=(NPAD // BLK,),
        in_specs=[
            pl.BlockSpec((BLK, D), lambda i: (i, 0)),
            pl.BlockSpec((D, D), lambda i: (0, 0)),
            pl.BlockSpec((1, D), lambda i: (0, 0)),
        ],
        out_specs=pl.BlockSpec((BLK, D), lambda i: (i, 0)),
        out_shape=jax.ShapeDtypeStruct((NPAD, D), jnp.float32),
    )(x_pad, w, b.reshape(1, D))


def _layer_pre_body(num0_ref, num1_ref, den0_ref, den1_ref, skipin_ref,
                    lns_ref, lnb_ref, wskip_ref, bskip_ref,
                    wl_ref, bl_ref, wr_ref, br_ref, segT_ref,
                    skip_ref, hs_ref, hr_ref):
    num = num0_ref[...] + num1_ref[...]
    den = jnp.dot(den0_ref[...] + den1_ref[...], segT_ref[...],
                  preferred_element_type=jnp.float32)
    gat = num / (den + 1e-9)
    x = jnp.maximum(gat + skipin_ref[...], 0.0)
    m = x.mean(-1, keepdims=True)
    v = jnp.mean((x - m) * (x - m), axis=-1, keepdims=True)
    x = (x - m) * jax.lax.rsqrt(v + 1e-6) * lns_ref[...] + lnb_ref[...]
    skip_ref[...] = jnp.dot(x, wskip_ref[...], preferred_element_type=jnp.float32) + bskip_ref[...]
    hs_ref[...] = jnp.dot(x, wl_ref[...], preferred_element_type=jnp.float32) + bl_ref[...]
    hr_ref[...] = jnp.dot(x, wr_ref[...], preferred_element_type=jnp.float32) + br_ref[...]


def _layer_pre(num, den_t, skipin_pad, lp):
    gp = lp['gat']
    den0 = den_t[0].reshape(NTAB, H)
    den1 = den_t[1].reshape(NTAB, H)
    segT = jnp.asarray(_SEG).T
    args = (num[0], num[1], den0, den1, skipin_pad,
            lp['ln_s'].reshape(1, D), lp['ln_b'].reshape(1, D),
            lp['skip_W'], lp['skip_b'].reshape(1, D),
            gp['Wl'], gp['bl'].reshape(1, D),
            gp['Wr'], gp['br'].reshape(1, D), segT)
    return pl.pallas_call(
        _layer_pre_body,
        grid=(NPAD // BLK,),
        in_specs=[
            pl.BlockSpec((BLK, D), lambda i: (i, 0)),
            pl.BlockSpec((BLK, D), lambda i: (i, 0)),
            pl.BlockSpec((BLK, H), lambda i: (i, 0)),
            pl.BlockSpec((BLK, H), lambda i: (i, 0)),
            pl.BlockSpec((BLK, D), lambda i: (i, 0)),
            pl.BlockSpec((1, D), lambda i: (0, 0)),
            pl.BlockSpec((1, D), lambda i: (0, 0)),
            pl.BlockSpec((D, D), lambda i: (0, 0)),
            pl.BlockSpec((1, D), lambda i: (0, 0)),
            pl.BlockSpec((D, D), lambda i: (0, 0)),
            pl.BlockSpec((1, D), lambda i: (0, 0)),
            pl.BlockSpec((D, D), lambda i: (0, 0)),
            pl.BlockSpec((1, D), lambda i: (0, 0)),
            pl.BlockSpec((H, D), lambda i: (0, 0)),
        ],
        out_specs=[pl.BlockSpec((BLK, D), lambda i: (i, 0))] * 3,
        out_shape=[jax.ShapeDtypeStruct((NPAD, D), jnp.float32)] * 3,
    )(*args)


EBLK = 1000


def _he_body(ef_ref, we_ref, be_ref, out_ref):
    out_ref[...] = jnp.dot(ef_ref[...], we_ref[...],
                           preferred_element_type=jnp.float32) + be_ref[...]


def _he_proj(edge_features, gp):
    return pl.pallas_call(
        _he_body,
        grid=(E // EBLK,),
        in_specs=[
            pl.BlockSpec((EBLK, DE), lambda i: (i, 0)),
            pl.BlockSpec((DE, D), lambda i: (0, 0)),
            pl.BlockSpec((1, D), lambda i: (0, 0)),
        ],
        out_specs=pl.BlockSpec((EBLK, D), lambda i: (i, 0)),
        out_shape=jax.ShapeDtypeStruct((E, D), jnp.float32),
    )(edge_features, gp['We'], gp['be'].reshape(1, D))


def _pre0_body(x_ref, lns_ref, lnb_ref, wskip_ref, bskip_ref,
               wl_ref, bl_ref, wr_ref, br_ref, skip_ref, hs_ref, hr_ref):
    x = x_ref[...]
    m = x.mean(-1, keepdims=True)
    v = jnp.mean((x - m) * (x - m), axis=-1, keepdims=True)
    x = (x - m) * jax.lax.rsqrt(v + 1e-6) * lns_ref[...] + lnb_ref[...]
    skip_ref[...] = jnp.dot(x, wskip_ref[...], preferred_element_type=jnp.float32) + bskip_ref[...]
    hs_ref[...] = jnp.dot(x, wl_ref[...], preferred_element_type=jnp.float32) + bl_ref[...]
    hr_ref[...] = jnp.dot(x, wr_ref[...], preferred_element_type=jnp.float32) + br_ref[...]


def _pre0(x_pad, lp):
    gp = lp['gat']
    return pl.pallas_call(
        _pre0_body,
        grid=(NPAD // BLK,),
        in_specs=[
            pl.BlockSpec((BLK, D), lambda i: (i, 0)),
            pl.BlockSpec((1, D), lambda i: (0, 0)),
            pl.BlockSpec((1, D), lambda i: (0, 0)),
            pl.BlockSpec((D, D), lambda i: (0, 0)),
            pl.BlockSpec((1, D), lambda i: (0, 0)),
            pl.BlockSpec((D, D), lambda i: (0, 0)),
            pl.BlockSpec((1, D), lambda i: (0, 0)),
            pl.BlockSpec((D, D), lambda i: (0, 0)),
            pl.BlockSpec((1, D), lambda i: (0, 0)),
        ],
        out_specs=[pl.BlockSpec((BLK, D), lambda i: (i, 0))] * 3,
        out_shape=[jax.ShapeDtypeStruct((NPAD, D), jnp.float32)] * 3,
    )(x_pad, lp['ln_s'].reshape(1, D), lp['ln_b'].reshape(1, D),
      lp['skip_W'], lp['skip_b'].reshape(1, D),
      gp['Wl'], gp['bl'].reshape(1, D), gp['Wr'], gp['br'].reshape(1, D))


def _gat_fin_body(num0_ref, num1_ref, den0_ref, den1_ref, skipin_ref,
                  segT_ref, out_ref):
    num = num0_ref[...] + num1_ref[...]
    den = jnp.dot(den0_ref[...] + den1_ref[...], segT_ref[...],
                  preferred_element_type=jnp.float32)
    out_ref[...] = jnp.maximum(num / (den + 1e-9) + skipin_ref[...], 0.0)


def _gat_fin(num, den_t, skipin_pad):
    den0 = den_t[0].reshape(NTAB, H)
    den1 = den_t[1].reshape(NTAB, H)
    segT = jnp.asarray(_SEG).T
    return pl.pallas_call(
        _gat_fin_body,
        grid=(NPAD // BLK,),
        in_specs=[
            pl.BlockSpec((BLK, D), lambda i: (i, 0)),
            pl.BlockSpec((BLK, D), lambda i: (i, 0)),
            pl.BlockSpec((BLK, H), lambda i: (i, 0)),
            pl.BlockSpec((BLK, H), lambda i: (i, 0)),
            pl.BlockSpec((BLK, D), lambda i: (i, 0)),
            pl.BlockSpec((H, D), lambda i: (0, 0)),
        ],
        out_specs=pl.BlockSpec((BLK, D), lambda i: (i, 0)),
        out_shape=jax.ShapeDtypeStruct((NPAD, D), jnp.float32),
    )(num[0], num[1], den0, den1, skipin_pad, segT)


def kernel(node_features, node_mask, edge_features, global_features, edge_list,
           edge_mask, params):
    p = params
    senders = edge_list[:, 0]
    receivers = edge_list[:, 1]
    n = node_features.shape[0]
    nf = jnp.concatenate([node_features, jnp.repeat(global_features, n, axis=0)],
                         axis=-1)
    nf = jnp.concatenate([nf, jnp.zeros((1, nf.shape[-1]), jnp.float32)], axis=0)
    nm_pad = jnp.concatenate([node_mask, jnp.zeros((NPAD - N,), jnp.float32)])
    snd_sc = jnp.where(edge_mask, senders, NTAB - 1).astype(jnp.int32)
    rcv_sc = jnp.where(edge_mask, receivers, NTAB - 1).astype(jnp.int32)
    g = jnp.tile(p['global'], (1, 1))

    nf_pad = jnp.pad(nf, ((0, NPAD - NP1), (0, 0)))
    nodes = _encoder(nf_pad, p)

    g = _attention_pallas(g, nodes, nm_pad, p['attn1'])
    mix_bias = (g @ p['mix_W'][D:] + p['mix_b']).reshape(1, D)
    nodes = _mm(nodes, p['mix_W'][:D], mix_bias, act='relu')

    skip, hs, hr = _pre0(nodes, p['layers'][0])
    for li, lp in enumerate(p['layers']):
        gp = lp['gat']
        he = _he_proj(edge_features, gp)
        att_flat = gp['att'].reshape(1, D)
        hs_tab = jnp.concatenate(
            [hs, att_flat, jnp.zeros((7, D), jnp.float32)], axis=0)
        num, den_t = _sc_edges(hs_tab, hr, he, snd_sc, rcv_sc)
        if li + 1 < L:
            skip, hs, hr = _layer_pre(num, den_t, skip, p['layers'][li + 1])
        else:
            nodes = _gat_fin(num, den_t, skip)
    g = _attention_pallas(g, nodes, nm_pad, p['attn2'])
    g = jax.nn.relu(_ln(g, p['final_ln_s'], p['final_ln_b']))
    return g.reshape(-1)


# speedup vs baseline: 1.0965x; 1.0965x over previous
import functools

import jax
import jax.numpy as jnp
import numpy as np
from jax import lax
from jax.experimental import pallas as pl
from jax.experimental.pallas import tpu as pltpu
from jax.experimental.pallas import tpu_sc as plsc

N = 10000
E = 320000
DFEAT = 128
DG = 32
DE = 16
D = 128
H = 8
DH = D // H
L = 3

NP1 = N + 1
BLK = 128
NPAD = ((NP1 + BLK - 1) // BLK) * BLK

NC = 2
NS = 16
NW = NC * NS
NTAB = 10112
RPT = NTAB // NS
EPW = E // NW
K = 80
DNR = NTAB // 16


def _ln(x, s, b):
    m = x.mean(-1, keepdims=True)
    v = x.var(-1, keepdims=True)
    return (x - m) / jnp.sqrt(v + 1e-6) * s + b


def _encoder_body(nf_ref, w1_ref, b1_ref, lns_ref, lnb_ref, w2_ref, b2_ref,
                  out_ref):
    x = nf_ref[...]
    h = jnp.dot(x, w1_ref[...], preferred_element_type=jnp.float32) + b1_ref[...]
    m = h.mean(-1, keepdims=True)
    v = jnp.mean((h - m) * (h - m), axis=-1, keepdims=True)
    h = (h - m) * jax.lax.rsqrt(v + 1e-6) * lns_ref[...] + lnb_ref[...]
    h = jnp.maximum(h, 0.0)
    out_ref[...] = jnp.dot(h, w2_ref[...], preferred_element_type=jnp.float32) + b2_ref[...]


def _encoder(nf_pad, p):
    grid = NPAD // BLK
    return pl.pallas_call(
        _encoder_body,
        grid=(grid,),
        in_specs=[
            pl.BlockSpec((BLK, DFEAT + DG), lambda i: (i, 0)),
            pl.BlockSpec((DFEAT + DG, D), lambda i: (0, 0)),
            pl.BlockSpec((D,), lambda i: (0,)),
            pl.BlockSpec((D,), lambda i: (0,)),
            pl.BlockSpec((D,), lambda i: (0,)),
            pl.BlockSpec((D, D), lambda i: (0, 0)),
            pl.BlockSpec((D,), lambda i: (0,)),
        ],
        out_specs=pl.BlockSpec((BLK, D), lambda i: (i, 0)),
        out_shape=jax.ShapeDtypeStruct((NPAD, D), jnp.float32),
    )(nf_pad, p['enc_W1'], p['enc_b1'], p['enc_ln_s'], p['enc_ln_b'],
      p['enc_W2'], p['enc_b2'])


def _sc_edge_body(hs_tab, hr_tab, he_hbm, snd, rcv,
                  out_num, out_den,
                  acc, den_acc, att_v, idx_s, idx_r, idx_r2,
                  hs_rows, hr_rows, he_rows, exbuf, staged_den, zbuf,
                  sem_s, sem_r, sem_e):
    cid = lax.axis_index("c")
    sid = lax.axis_index("s")
    wid = cid * NS + sid
    zero16 = jnp.zeros((16,), jnp.float32)
    iota16 = lax.iota(jnp.int32, 16)

    def zrow(r, carry):
        for c in range(8):
            zbuf[r, pl.ds(c * 16, 16)] = zero16
        return carry
    lax.fori_loop(0, 8, zrow, 0)

    def zacc(t, carry):
        pltpu.sync_copy(zbuf, acc.at[pl.ds(sid * RPT + t * 8, 8)])
        return carry
    lax.fori_loop(0, RPT // 8, zacc, 0)
    dstart = jnp.minimum(sid * 40, DNR - 40)
    for t in range(5):
        pltpu.sync_copy(zbuf, den_acc.at[pl.ds(dstart + t * 8, 8)])

    def zsd(r, carry):
        for c in range(8):
            staged_den[r, pl.ds(c * 16, 16)] = zero16
        return carry
    lax.fori_loop(0, K, zsd, 0)
    pltpu.sync_copy(hs_tab.at[pl.ds(NTAB, 1)], att_v)
    plsc.subcore_barrier()

    ebase = wid * EPW

    def batch(b, carry):
        off = ebase + b * K
        pltpu.sync_copy(snd.at[pl.ds(off, K)], idx_s)
        pltpu.sync_copy(rcv.at[pl.ds(off, K)], idx_r)
        cs = pltpu.async_copy(hs_tab.at[idx_s], hs_rows, sem_s)
        cr = pltpu.async_copy(hr_tab.at[idx_r], hr_rows, sem_r)
        ce = pltpu.async_copy(he_hbm.at[pl.ds(off, K)], he_rows, sem_e)
        cs.wait()
        cr.wait()
        ce.wait()

        def p1(k, c1):
            for j in range(H):
                sl = pl.ds(j * 16, 16)
                mv = hs_rows[k, sl] + hr_rows[k, sl] + he_rows[k, sl]
                he_rows[k, sl] = jnp.maximum(mv, mv * 0.2)
            return c1
        lax.fori_loop(0, K, p1, 0, unroll=4)

        def p2(g, c2):
            rows = g * 16 + iota16
            rvec = idx_r[pl.ds(g * 16, 16)]
            idx_r2[pl.ds(g * 16, 16)] = lax.shift_right_logical(rvec, 4)
            posv = (rvec & 15) * 8
            for h in range(H):
                attv = att_v[0, pl.ds(h * 16, 16)]
                lg = jnp.zeros((16,), jnp.float32)
                for dd in range(16):
                    col = plsc.load_gather(
                        he_rows, [rows, jnp.full((16,), h * 16 + dd, jnp.int32)])
                    lg = lg + col * attv[dd]
                exh = jnp.exp(lg)
                plsc.store_scatter(exbuf, [g * 256 + iota16 * 16 + h], exh)
                plsc.store_scatter(staged_den, [rows, posv + h], exh)
            return c2
        lax.fori_loop(0, K // 16, p2, 0)

        def p3(k, c3):
            exrow = exbuf[pl.ds(k * 16, 16)]
            for j in range(H):
                sl = pl.ds(j * 16, 16)
                hr_rows[k, sl] = hs_rows[k, sl] * exrow[j]
            return c3
        lax.fori_loop(0, K, p3, 0, unroll=4)

        a1 = pltpu.async_copy(hr_rows, acc.at[idx_r], sem_s, add=True)
        a2 = pltpu.async_copy(staged_den, den_acc.at[idx_r2], sem_r, add=True)
        a2.wait()

        def pc(g, c4):
            rows = g * 16 + iota16
            posv = (idx_r[pl.ds(g * 16, 16)] & 15) * 8
            for h in range(H):
                plsc.store_scatter(staged_den, [rows, posv + h], zero16)
            return c4
        lax.fori_loop(0, K // 16, pc, 0)
        a1.wait()
        return carry
    lax.fori_loop(0, EPW // K, batch, 0)

    plsc.subcore_barrier()
    pltpu.sync_copy(acc.at[pl.ds(sid * RPT, RPT)],
                    out_num.at[cid, pl.ds(sid * RPT, RPT)])
    dstart2 = jnp.minimum(sid * 40, DNR - 40)
    pltpu.sync_copy(den_acc.at[pl.ds(dstart2, 40)],
                    out_den.at[cid, pl.ds(dstart2, 40)])


_sc_edges_built = None


def _build_sc_edges():
    return pl.kernel(
        _sc_edge_body,
        out_type=(jax.ShapeDtypeStruct((NC, NTAB, D), jnp.float32),
                  jax.ShapeDtypeStruct((NC, DNR, D), jnp.float32)),
        mesh=plsc.VectorSubcoreMesh(core_axis_name="c", subcore_axis_name="s",
                                    num_cores=NC, num_subcores=NS),
        scratch_types=[
            pltpu.VMEM_SHARED((NTAB, D), jnp.float32),
            pltpu.VMEM_SHARED((DNR, D), jnp.float32),
            pltpu.VMEM((1, D), jnp.float32),
            pltpu.VMEM((K,), jnp.int32),
            pltpu.VMEM((K,), jnp.int32),
            pltpu.VMEM((K,), jnp.int32),
            pltpu.VMEM((K, D), jnp.float32),
            pltpu.VMEM((K, D), jnp.float32),
            pltpu.VMEM((K, D), jnp.float32),
            pltpu.VMEM((K * 16,), jnp.float32),
            pltpu.VMEM((K, D), jnp.float32),
            pltpu.VMEM((8, D), jnp.float32),
            pltpu.SemaphoreType.DMA,
            pltpu.SemaphoreType.DMA,
            pltpu.SemaphoreType.DMA,
        ],
        compiler_params=pltpu.CompilerParams(needs_layout_passes=False),
        name="sc_gatv2_edges",
    )


def _sc_edges(*args):
    global _sc_edges_built
    if _sc_edges_built is None:
        _sc_edges_built = _build_sc_edges()
    return _sc_edges_built(*args)


_SEG = np.kron(np.eye(H, dtype=np.float32), np.ones((DH, 1), np.float32))


_P = np.kron(np.eye(H, dtype=np.float32), np.ones((DH, DH), np.float32))


def _attn_body(nodes_ref, nm_ref, wk_ref, bk_ref, wv_ref, bv_ref,
               klns_ref, klnb_ref, q_ref, pmat_ref,
               out_ref, macc_ref, sacc_ref, vacc_ref):
    i = pl.program_id(0)

    @pl.when(i == 0)
    def _init():
        macc_ref[...] = jnp.full((1, D), -1e30, jnp.float32)
        sacc_ref[...] = jnp.zeros((1, D), jnp.float32)
        vacc_ref[...] = jnp.zeros((1, D), jnp.float32)

    x = nodes_ref[...]
    pmat = pmat_ref[...]
    k = jnp.dot(x, wk_ref[...], preferred_element_type=jnp.float32) + bk_ref[...]
    mfull = jnp.dot(k, pmat, preferred_element_type=jnp.float32) * (1.0 / DH)
    kc = k - mfull
    vfull = jnp.dot(kc * kc, pmat, preferred_element_type=jnp.float32) * (1.0 / DH)
    kln = kc * jax.lax.rsqrt(vfull + 1e-6) * klns_ref[...] + klnb_ref[...]
    sfull = jnp.dot(kln * q_ref[...], pmat,
                    preferred_element_type=jnp.float32) * (1.0 / np.sqrt(DH))
    sfull = jnp.where(nm_ref[...] > 0.0, sfull, -1e9)
    bm = jnp.max(sfull, axis=0, keepdims=True)
    m_old = macc_ref[...]
    m_new = jnp.maximum(m_old, bm)
    corr = jnp.exp(m_old - m_new)
    w = jnp.exp(sfull - m_new)
    v = jnp.dot(x, wv_ref[...], preferred_element_type=jnp.float32) + bv_ref[...]
    macc_ref[...] = m_new
    sacc_ref[...] = sacc_ref[...] * corr + jnp.sum(w, axis=0, keepdims=True)
    vacc_ref[...] = vacc_ref[...] * corr + jnp.sum(w * v, axis=0, keepdims=True)

    @pl.when(i == pl.num_programs(0) - 1)
    def _fin():
        out_ref[...] = vacc_ref[...] / sacc_ref[...]


def _attention_pallas(g, nodes_pad, nm_pad, ap):
    q = (g @ ap['Wq'] + ap['bq']).reshape(1, H, DH)
    q = _ln(q, ap['qln_s'], ap['qln_b']).reshape(1, D)
    klns = jnp.tile(ap['kln_s'], H).reshape(1, D)
    klnb = jnp.tile(ap['kln_b'], H).reshape(1, D)
    out = pl.pallas_call(
        _attn_body,
        grid=(NPAD // BLK,),
        in_specs=[
            pl.BlockSpec((BLK, D), lambda i: (i, 0)),
            pl.BlockSpec((BLK, D), lambda i: (i, 0)),
            pl.BlockSpec((D, D), lambda i: (0, 0)),
            pl.BlockSpec((D,), lambda i: (0,)),
            pl.BlockSpec((D, D), lambda i: (0, 0)),
            pl.BlockSpec((D,), lambda i: (0,)),
            pl.BlockSpec((1, D), lambda i: (0, 0)),
            pl.BlockSpec((1, D), lambda i: (0, 0)),
            pl.BlockSpec((1, D), lambda i: (0, 0)),
            pl.BlockSpec((D, D), lambda i: (0, 0)),
        ],
        out_specs=pl.BlockSpec((1, D), lambda i: (0, 0)),
        out_shape=jax.ShapeDtypeStruct((1, D), jnp.float32),
        scratch_shapes=[
            pltpu.VMEM((1, D), jnp.float32),
            pltpu.VMEM((1, D), jnp.float32),
            pltpu.VMEM((1, D), jnp.float32),
        ],
    )(nodes_pad, jnp.repeat(nm_pad[:, None], D, axis=1),
      ap['Wk'], ap['bk'], ap['Wv'], ap['bv'],
      klns, klnb, q, jnp.asarray(_P))
    return g + out @ ap['Wo'] + ap['bo']


def _mm_ln_body(x_ref, w_ref, b_ref, out_ref, *, act):
    h = jnp.dot(x_ref[...], w_ref[...], preferred_element_type=jnp.float32) + b_ref[...]
    if act == 'relu':
        h = jnp.maximum(h, 0.0)
    out_ref[...] = h


def _mm(x_pad, w, b, act='none'):
    return pl.pallas_call(
        functools.partial(_mm_ln_body, act=act),
        grid=(NPAD // BLK,),
        in_specs=[
            pl.BlockSpec((BLK, D), lambda i: (i, 0)),
            pl.BlockSpec((D, D), lambda i: (0, 0)),
            pl.BlockSpec((1, D), lambda i: (0, 0)),
        ],
        out_specs=pl.BlockSpec((BLK, D), lambda i: (i, 0)),
        out_shape=jax.ShapeDtypeStruct((NPAD, D), jnp.float32),
    )(x_pad, w, b.reshape(1, D))


def _layer_pre_body(num0_ref, num1_ref, den0_ref, den1_ref, skipin_ref,
                    lns_ref, lnb_ref, wskip_ref, bskip_ref,
                    wl_ref, bl_ref, wr_ref, br_ref, segT_ref,
                    skip_ref, hs_ref, hr_ref):
    num = num0_ref[...] + num1_ref[...]
    den = jnp.dot(den0_ref[...] + den1_ref[...], segT_ref[...],
                  preferred_element_type=jnp.float32)
    gat = num / (den + 1e-9)
    x = jnp.maximum(gat + skipin_ref[...], 0.0)
    m = x.mean(-1, keepdims=True)
    v = jnp.mean((x - m) * (x - m), axis=-1, keepdims=True)
    x = (x - m) * jax.lax.rsqrt(v + 1e-6) * lns_ref[...] + lnb_ref[...]
    skip_ref[...] = jnp.dot(x, wskip_ref[...], preferred_element_type=jnp.float32) + bskip_ref[...]
    hs_ref[...] = jnp.dot(x, wl_ref[...], preferred_element_type=jnp.float32) + bl_ref[...]
    hr_ref[...] = jnp.dot(x, wr_ref[...], preferred_element_type=jnp.float32) + br_ref[...]


def _layer_pre(num, den_t, skipin_pad, lp):
    gp = lp['gat']
    den0 = den_t[0].reshape(NTAB, H)
    den1 = den_t[1].reshape(NTAB, H)
    segT = jnp.asarray(_SEG).T
    args = (num[0], num[1], den0, den1, skipin_pad,
            lp['ln_s'].reshape(1, D), lp['ln_b'].reshape(1, D),
            lp['skip_W'], lp['skip_b'].reshape(1, D),
            gp['Wl'], gp['bl'].reshape(1, D),
            gp['Wr'], gp['br'].reshape(1, D), segT)
    return pl.pallas_call(
        _layer_pre_body,
        grid=(NPAD // BLK,),
        in_specs=[
            pl.BlockSpec((BLK, D), lambda i: (i, 0)),
            pl.BlockSpec((BLK, D), lambda i: (i, 0)),
            pl.BlockSpec((BLK, H), lambda i: (i, 0)),
            pl.BlockSpec((BLK, H), lambda i: (i, 0)),
            pl.BlockSpec((BLK, D), lambda i: (i, 0)),
            pl.BlockSpec((1, D), lambda i: (0, 0)),
            pl.BlockSpec((1, D), lambda i: (0, 0)),
            pl.BlockSpec((D, D), lambda i: (0, 0)),
            pl.BlockSpec((1, D), lambda i: (0, 0)),
            pl.BlockSpec((D, D), lambda i: (0, 0)),
            pl.BlockSpec((1, D), lambda i: (0, 0)),
            pl.BlockSpec((D, D), lambda i: (0, 0)),
            pl.BlockSpec((1, D), lambda i: (0, 0)),
            pl.BlockSpec((H, D), lambda i: (0, 0)),
        ],
        out_specs=[pl.BlockSpec((BLK, D), lambda i: (i, 0))] * 3,
        out_shape=[jax.ShapeDtypeStruct((NPAD, D), jnp.float32)] * 3,
    )(*args)


EBLK = 1000


def _he_body(ef_ref, we_ref, be_ref, out_ref):
    out_ref[...] = jnp.dot(ef_ref[...], we_ref[...],
                           preferred_element_type=jnp.float32) + be_ref[...]


def _he_proj(edge_features, gp):
    return pl.pallas_call(
        _he_body,
        grid=(E // EBLK,),
        in_specs=[
            pl.BlockSpec((EBLK, DE), lambda i: (i, 0)),
            pl.BlockSpec((DE, D), lambda i: (0, 0)),
            pl.BlockSpec((1, D), lambda i: (0, 0)),
        ],
        out_specs=pl.BlockSpec((EBLK, D), lambda i: (i, 0)),
        out_shape=jax.ShapeDtypeStruct((E, D), jnp.float32),
    )(edge_features, gp['We'], gp['be'].reshape(1, D))


def _pre0_body(x_ref, lns_ref, lnb_ref, wskip_ref, bskip_ref,
               wl_ref, bl_ref, wr_ref, br_ref, skip_ref, hs_ref, hr_ref):
    x = x_ref[...]
    m = x.mean(-1, keepdims=True)
    v = jnp.mean((x - m) * (x - m), axis=-1, keepdims=True)
    x = (x - m) * jax.lax.rsqrt(v + 1e-6) * lns_ref[...] + lnb_ref[...]
    skip_ref[...] = jnp.dot(x, wskip_ref[...], preferred_element_type=jnp.float32) + bskip_ref[...]
    hs_ref[...] = jnp.dot(x, wl_ref[...], preferred_element_type=jnp.float32) + bl_ref[...]
    hr_ref[...] = jnp.dot(x, wr_ref[...], preferred_element_type=jnp.float32) + br_ref[...]


def _pre0(x_pad, lp):
    gp = lp['gat']
    return pl.pallas_call(
        _pre0_body,
        grid=(NPAD // BLK,),
        in_specs=[
            pl.BlockSpec((BLK, D), lambda i: (i, 0)),
            pl.BlockSpec((1, D), lambda i: (0, 0)),
            pl.BlockSpec((1, D), lambda i: (0, 0)),
            pl.BlockSpec((D, D), lambda i: (0, 0)),
            pl.BlockSpec((1, D), lambda i: (0, 0)),
            pl.BlockSpec((D, D), lambda i: (0, 0)),
            pl.BlockSpec((1, D), lambda i: (0, 0)),
            pl.BlockSpec((D, D), lambda i: (0, 0)),
            pl.BlockSpec((1, D), lambda i: (0, 0)),
        ],
        out_specs=[pl.BlockSpec((BLK, D), lambda i: (i, 0))] * 3,
        out_shape=[jax.ShapeDtypeStruct((NPAD, D), jnp.float32)] * 3,
    )(x_pad, lp['ln_s'].reshape(1, D), lp['ln_b'].reshape(1, D),
      lp['skip_W'], lp['skip_b'].reshape(1, D),
      gp['Wl'], gp['bl'].reshape(1, D), gp['Wr'], gp['br'].reshape(1, D))


def _gat_fin_body(num0_ref, num1_ref, den0_ref, den1_ref, skipin_ref,
                  segT_ref, out_ref):
    num = num0_ref[...] + num1_ref[...]
    den = jnp.dot(den0_ref[...] + den1_ref[...], segT_ref[...],
                  preferred_element_type=jnp.float32)
    out_ref[...] = jnp.maximum(num / (den + 1e-9) + skipin_ref[...], 0.0)


def _gat_fin(num, den_t, skipin_pad):
    den0 = den_t[0].reshape(NTAB, H)
    den1 = den_t[1].reshape(NTAB, H)
    segT = jnp.asarray(_SEG).T
    return pl.pallas_call(
        _gat_fin_body,
        grid=(NPAD // BLK,),
        in_specs=[
            pl.BlockSpec((BLK, D), lambda i: (i, 0)),
            pl.BlockSpec((BLK, D), lambda i: (i, 0)),
            pl.BlockSpec((BLK, H), lambda i: (i, 0)),
            pl.BlockSpec((BLK, H), lambda i: (i, 0)),
            pl.BlockSpec((BLK, D), lambda i: (i, 0)),
            pl.BlockSpec((H, D), lambda i: (0, 0)),
        ],
        out_specs=pl.BlockSpec((BLK, D), lambda i: (i, 0)),
        out_shape=jax.ShapeDtypeStruct((NPAD, D), jnp.float32),
    )(num[0], num[1], den0, den1, skipin_pad, segT)


def kernel(node_features, node_mask, edge_features, global_features, edge_list,
           edge_mask, params):
    p = params
    senders = edge_list[:, 0]
    receivers = edge_list[:, 1]
    n = node_features.shape[0]
    nf = jnp.concatenate([node_features, jnp.repeat(global_features, n, axis=0)],
                         axis=-1)
    nf = jnp.concatenate([nf, jnp.zeros((1, nf.shape[-1]), jnp.float32)], axis=0)
    nm_pad = jnp.concatenate([node_mask, jnp.zeros((NPAD - N,), jnp.float32)])
    snd_sc = jnp.where(edge_mask, senders, NTAB - 1).astype(jnp.int32)
    rcv_sc = jnp.where(edge_mask, receivers, NTAB - 1).astype(jnp.int32)
    g = jnp.tile(p['global'], (1, 1))

    nf_pad = jnp.pad(nf, ((0, NPAD - NP1), (0, 0)))
    nodes = _encoder(nf_pad, p)

    g = _attention_pallas(g, nodes, nm_pad, p['attn1'])
    mix_bias = (g @ p['mix_W'][D:] + p['mix_b']).reshape(1, D)
    nodes = _mm(nodes, p['mix_W'][:D], mix_bias, act='relu')

    skip, hs, hr = _pre0(nodes, p['layers'][0])
    for li, lp in enumerate(p['layers']):
        gp = lp['gat']
        he = _he_proj(edge_features, gp)
        att_flat = gp['att'].reshape(1, D)
        hs_tab = jnp.concatenate(
            [hs, att_flat, jnp.zeros((7, D), jnp.float32)], axis=0)
        num, den_t = _sc_edges(hs_tab, hr, he, snd_sc, rcv_sc)
        if li + 1 < L:
            skip, hs, hr = _layer_pre(num, den_t, skip, p['layers'][li + 1])
        else:
            nodes = _gat_fin(num, den_t, skip)
    g = _attention_pallas(g, nodes, nm_pad, p['attn2'])
    g = jax.nn.relu(_ln(g, p['final_ln_s'], p['final_ln_b']))
    return g.reshape(-1)


# speedup vs baseline: 1.6212x; 1.4785x over previous
import functools

import jax
import jax.numpy as jnp
import numpy as np
from jax import lax
from jax.experimental import pallas as pl
from jax.experimental.pallas import tpu as pltpu
from jax.experimental.pallas import tpu_sc as plsc

N = 10000
E = 320000
DFEAT = 128
DG = 32
DE = 16
D = 128
H = 8
DH = D // H
L = 3

NP1 = N + 1
BLK = 128
NPAD = ((NP1 + BLK - 1) // BLK) * BLK

NC = 2
NS = 16
NW = NC * NS
NTAB = 10112
RPT = NTAB // NS
EPW = E // NW
K = 80
DNR = NTAB // 16


def _ln(x, s, b):
    m = x.mean(-1, keepdims=True)
    v = x.var(-1, keepdims=True)
    return (x - m) / jnp.sqrt(v + 1e-6) * s + b


def _encoder_body(nf_ref, w1_ref, b1_ref, lns_ref, lnb_ref, w2_ref, b2_ref,
                  out_ref):
    x = nf_ref[...]
    h = jnp.dot(x, w1_ref[...], preferred_element_type=jnp.float32) + b1_ref[...]
    m = h.mean(-1, keepdims=True)
    v = jnp.mean((h - m) * (h - m), axis=-1, keepdims=True)
    h = (h - m) * jax.lax.rsqrt(v + 1e-6) * lns_ref[...] + lnb_ref[...]
    h = jnp.maximum(h, 0.0)
    out_ref[...] = jnp.dot(h, w2_ref[...], preferred_element_type=jnp.float32) + b2_ref[...]


def _encoder(nf_pad, p):
    grid = NPAD // BLK
    return pl.pallas_call(
        _encoder_body,
        grid=(grid,),
        in_specs=[
            pl.BlockSpec((BLK, DFEAT + DG), lambda i: (i, 0)),
            pl.BlockSpec((DFEAT + DG, D), lambda i: (0, 0)),
            pl.BlockSpec((D,), lambda i: (0,)),
            pl.BlockSpec((D,), lambda i: (0,)),
            pl.BlockSpec((D,), lambda i: (0,)),
            pl.BlockSpec((D, D), lambda i: (0, 0)),
            pl.BlockSpec((D,), lambda i: (0,)),
        ],
        out_specs=pl.BlockSpec((BLK, D), lambda i: (i, 0)),
        out_shape=jax.ShapeDtypeStruct((NPAD, D), jnp.float32),
    )(nf_pad, p['enc_W1'], p['enc_b1'], p['enc_ln_s'], p['enc_ln_b'],
      p['enc_W2'], p['enc_b2'])


def _sc_edge_body(hs_tab, hr_tab, he_hbm, snd, rcv,
                  out_num, out_den,
                  acc, den_acc, att_v, idx_s, idx_r, idx_r2,
                  hs_rows, hr_rows, he_rows, exbuf, staged_den, zbuf,
                  sem_s, sem_r, sem_e):
    cid = lax.axis_index("c")
    sid = lax.axis_index("s")
    wid = cid * NS + sid
    zero16 = jnp.zeros((16,), jnp.float32)
    iota16 = lax.iota(jnp.int32, 16)

    def zrow(r, carry):
        for c in range(8):
            zbuf[r, pl.ds(c * 16, 16)] = zero16
        return carry
    lax.fori_loop(0, 8, zrow, 0)

    def zacc(t, carry):
        pltpu.sync_copy(zbuf, acc.at[pl.ds(sid * RPT + t * 8, 8)])
        return carry
    lax.fori_loop(0, RPT // 8, zacc, 0)
    dstart = jnp.minimum(sid * 40, DNR - 40)
    for t in range(5):
        pltpu.sync_copy(zbuf, den_acc.at[pl.ds(dstart + t * 8, 8)])

    def zsd(r, carry):
        for c in range(8):
            staged_den[r, pl.ds(c * 16, 16)] = zero16
        return carry
    lax.fori_loop(0, K, zsd, 0)
    pltpu.sync_copy(hs_tab.at[pl.ds(NTAB, 1)], att_v)
    plsc.subcore_barrier()

    ebase = wid * EPW

    def batch(b, carry):
        off = ebase + b * K
        pltpu.sync_copy(snd.at[pl.ds(off, K)], idx_s)
        pltpu.sync_copy(rcv.at[pl.ds(off, K)], idx_r)
        cs = pltpu.async_copy(hs_tab.at[idx_s], hs_rows, sem_s)
        cr = pltpu.async_copy(hr_tab.at[idx_r], hr_rows, sem_r)
        ce = pltpu.async_copy(he_hbm.at[pl.ds(off, K)], he_rows, sem_e)
        cs.wait()
        cr.wait()
        ce.wait()

        def p1(k, c1):
            for j in range(H):
                sl = pl.ds(j * 16, 16)
                mv = hs_rows[k, sl] + hr_rows[k, sl] + he_rows[k, sl]
                he_rows[k, sl] = jnp.maximum(mv, mv * 0.2)
            return c1
        lax.fori_loop(0, K, p1, 0)

        def p2(g, c2):
            rows = g * 16 + iota16
            rvec = idx_r[pl.ds(g * 16, 16)]
            idx_r2[pl.ds(g * 16, 16)] = lax.shift_right_logical(rvec, 4)
            posv = (rvec & 15) * 8
            for h in range(H):
                attv = att_v[0, pl.ds(h * 16, 16)]
                lg = jnp.zeros((16,), jnp.float32)
                for dd in range(16):
                    col = plsc.load_gather(
                        he_rows, [rows, jnp.full((16,), h * 16 + dd, jnp.int32)])
                    lg = lg + col * attv[dd]
                exh = jnp.exp(lg)
                plsc.store_scatter(exbuf, [g * 256 + iota16 * 16 + h], exh)
                plsc.store_scatter(staged_den, [rows, posv + h], exh)
            return c2
        lax.fori_loop(0, K // 16, p2, 0)

        def p3(k, c3):
            exrow = exbuf[pl.ds(k * 16, 16)]
            for j in range(H):
                sl = pl.ds(j * 16, 16)
                hr_rows[k, sl] = hs_rows[k, sl] * exrow[j]
            return c3
        lax.fori_loop(0, K, p3, 0)

        a1 = pltpu.async_copy(hr_rows, acc.at[idx_r], sem_s, add=True)
        a2 = pltpu.async_copy(staged_den, den_acc.at[idx_r2], sem_r, add=True)
        a2.wait()

        def pc(g, c4):
            rows = g * 16 + iota16
            posv = (idx_r[pl.ds(g * 16, 16)] & 15) * 8
            for h in range(H):
                plsc.store_scatter(staged_den, [rows, posv + h], zero16)
            return c4
        lax.fori_loop(0, K // 16, pc, 0)
        a1.wait()
        return carry
    lax.fori_loop(0, EPW // K, batch, 0)

    plsc.subcore_barrier()
    pltpu.sync_copy(acc.at[pl.ds(sid * RPT, RPT)],
                    out_num.at[cid, pl.ds(sid * RPT, RPT)])
    dstart2 = jnp.minimum(sid * 40, DNR - 40)
    pltpu.sync_copy(den_acc.at[pl.ds(dstart2, 40)],
                    out_den.at[cid, pl.ds(dstart2, 40)])


_sc_edges_built = None


def _build_sc_edges():
    return pl.kernel(
        _sc_edge_body,
        out_type=(jax.ShapeDtypeStruct((NC, NTAB, D), jnp.float32),
                  jax.ShapeDtypeStruct((NC, DNR, D), jnp.float32)),
        mesh=plsc.VectorSubcoreMesh(core_axis_name="c", subcore_axis_name="s",
                                    num_cores=NC, num_subcores=NS),
        scratch_types=[
            pltpu.VMEM_SHARED((NTAB, D), jnp.float32),
            pltpu.VMEM_SHARED((DNR, D), jnp.float32),
            pltpu.VMEM((1, D), jnp.float32),
            pltpu.VMEM((K,), jnp.int32),
            pltpu.VMEM((K,), jnp.int32),
            pltpu.VMEM((K,), jnp.int32),
            pltpu.VMEM((K, D), jnp.float32),
            pltpu.VMEM((K, D), jnp.float32),
            pltpu.VMEM((K, D), jnp.float32),
            pltpu.VMEM((K * 16,), jnp.float32),
            pltpu.VMEM((K, D), jnp.float32),
            pltpu.VMEM((8, D), jnp.float32),
            pltpu.SemaphoreType.DMA,
            pltpu.SemaphoreType.DMA,
            pltpu.SemaphoreType.DMA,
        ],
        compiler_params=pltpu.CompilerParams(needs_layout_passes=False),
        name="sc_gatv2_edges",
    )


def _sc_edges(*args):
    global _sc_edges_built
    if _sc_edges_built is None:
        _sc_edges_built = _build_sc_edges()
    return _sc_edges_built(*args)


_SEG = np.kron(np.eye(H, dtype=np.float32), np.ones((DH, 1), np.float32))


_P = np.kron(np.eye(H, dtype=np.float32), np.ones((DH, DH), np.float32))


def _attn_body(nodes_ref, nm_ref, wk_ref, bk_ref, wv_ref, bv_ref,
               klns_ref, klnb_ref, q_ref, pmat_ref,
               out_ref, macc_ref, sacc_ref, vacc_ref):
    i = pl.program_id(0)

    @pl.when(i == 0)
    def _init():
        macc_ref[...] = jnp.full((1, D), -1e30, jnp.float32)
        sacc_ref[...] = jnp.zeros((1, D), jnp.float32)
        vacc_ref[...] = jnp.zeros((1, D), jnp.float32)

    x = nodes_ref[...]
    pmat = pmat_ref[...]
    k = jnp.dot(x, wk_ref[...], preferred_element_type=jnp.float32) + bk_ref[...]
    mfull = jnp.dot(k, pmat, preferred_element_type=jnp.float32) * (1.0 / DH)
    kc = k - mfull
    vfull = jnp.dot(kc * kc, pmat, preferred_element_type=jnp.float32) * (1.0 / DH)
    kln = kc * jax.lax.rsqrt(vfull + 1e-6) * klns_ref[...] + klnb_ref[...]
    sfull = jnp.dot(kln * q_ref[...], pmat,
                    preferred_element_type=jnp.float32) * (1.0 / np.sqrt(DH))
    sfull = jnp.where(nm_ref[...] > 0.0, sfull, -1e9)
    bm = jnp.max(sfull, axis=0, keepdims=True)
    m_old = macc_ref[...]
    m_new = jnp.maximum(m_old, bm)
    corr = jnp.exp(m_old - m_new)
    w = jnp.exp(sfull - m_new)
    v = jnp.dot(x, wv_ref[...], preferred_element_type=jnp.float32) + bv_ref[...]
    macc_ref[...] = m_new
    sacc_ref[...] = sacc_ref[...] * corr + jnp.sum(w, axis=0, keepdims=True)
    vacc_ref[...] = vacc_ref[...] * corr + jnp.sum(w * v, axis=0, keepdims=True)

    @pl.when(i == pl.num_programs(0) - 1)
    def _fin():
        out_ref[...] = vacc_ref[...] / sacc_ref[...]


def _attention_pallas(g, nodes_pad, nm_pad, ap):
    q = (g @ ap['Wq'] + ap['bq']).reshape(1, H, DH)
    q = _ln(q, ap['qln_s'], ap['qln_b']).reshape(1, D)
    klns = jnp.tile(ap['kln_s'], H).reshape(1, D)
    klnb = jnp.tile(ap['kln_b'], H).reshape(1, D)
    out = pl.pallas_call(
        _attn_body,
        grid=(NPAD // BLK,),
        in_specs=[
            pl.BlockSpec((BLK, D), lambda i: (i, 0)),
            pl.BlockSpec((BLK, D), lambda i: (i, 0)),
            pl.BlockSpec((D, D), lambda i: (0, 0)),
            pl.BlockSpec((D,), lambda i: (0,)),
            pl.BlockSpec((D, D), lambda i: (0, 0)),
            pl.BlockSpec((D,), lambda i: (0,)),
            pl.BlockSpec((1, D), lambda i: (0, 0)),
            pl.BlockSpec((1, D), lambda i: (0, 0)),
            pl.BlockSpec((1, D), lambda i: (0, 0)),
            pl.BlockSpec((D, D), lambda i: (0, 0)),
        ],
        out_specs=pl.BlockSpec((1, D), lambda i: (0, 0)),
        out_shape=jax.ShapeDtypeStruct((1, D), jnp.float32),
        scratch_shapes=[
            pltpu.VMEM((1, D), jnp.float32),
            pltpu.VMEM((1, D), jnp.float32),
            pltpu.VMEM((1, D), jnp.float32),
        ],
    )(nodes_pad, jnp.repeat(nm_pad[:, None], D, axis=1),
      ap['Wk'], ap['bk'], ap['Wv'], ap['bv'],
      klns, klnb, q, jnp.asarray(_P))
    return g + out @ ap['Wo'] + ap['bo']


def _mm_ln_body(x_ref, w_ref, b_ref, out_ref, *, act):
    h = jnp.dot(x_ref[...], w_ref[...], preferred_element_type=jnp.float32) + b_ref[...]
    if act == 'relu':
        h = jnp.maximum(h, 0.0)
    out_ref[...] = h


def _mm(x_pad, w, b, act='none'):
    return pl.pallas_call(
        functools.partial(_mm_ln_body, act=act),
        grid=(NPAD // BLK,),
        in_specs=[
            pl.BlockSpec((BLK, D), lambda i: (i, 0)),
            pl.BlockSpec((D, D), lambda i: (0, 0)),
            pl.BlockSpec((1, D), lambda i: (0, 0)),
        ],
        out_specs=pl.BlockSpec((BLK, D), lambda i: (i, 0)),
        out_shape=jax.ShapeDtypeStruct((NPAD, D), jnp.float32),
    )(x_pad, w, b.reshape(1, D))


def _layer_pre_body(num0_ref, num1_ref, den0_ref, den1_ref, skipin_ref,
                    lns_ref, lnb_ref, wskip_ref, bskip_ref,
                    wl_ref, bl_ref, wr_ref, br_ref, segT_ref,
                    skip_ref, hs_ref, hr_ref):
    num = num0_ref[...] + num1_ref[...]
    den = jnp.dot(den0_ref[...] + den1_ref[...], segT_ref[...],
                  preferred_element_type=jnp.float32)
    gat = num / (den + 1e-9)
    x = jnp.maximum(gat + skipin_ref[...], 0.0)
    m = x.mean(-1, keepdims=True)
    v = jnp.mean((x - m) * (x - m), axis=-1, keepdims=True)
    x = (x - m) * jax.lax.rsqrt(v + 1e-6) * lns_ref[...] + lnb_ref[...]
    skip_ref[...] = jnp.dot(x, wskip_ref[...], preferred_element_type=jnp.float32) + bskip_ref[...]
    hs_ref[...] = jnp.dot(x, wl_ref[...], preferred_element_type=jnp.float32) + bl_ref[...]
    hr_ref[...] = jnp.dot(x, wr_ref[...], preferred_element_type=jnp.float32) + br_ref[...]


def _layer_pre(num, den_t, skipin_pad, lp):
    gp = lp['gat']
    den0 = den_t[0].reshape(NTAB, H)
    den1 = den_t[1].reshape(NTAB, H)
    segT = jnp.asarray(_SEG).T
    args = (num[0], num[1], den0, den1, skipin_pad,
            lp['ln_s'].reshape(1, D), lp['ln_b'].reshape(1, D),
            lp['skip_W'], lp['skip_b'].reshape(1, D),
            gp['Wl'], gp['bl'].reshape(1, D),
            gp['Wr'], gp['br'].reshape(1, D), segT)
    return pl.pallas_call(
        _layer_pre_body,
        grid=(NPAD // BLK,),
        in_specs=[
            pl.BlockSpec((BLK, D), lambda i: (i, 0)),
            pl.BlockSpec((BLK, D), lambda i: (i, 0)),
            pl.BlockSpec((BLK, H), lambda i: (i, 0)),
            pl.BlockSpec((BLK, H), lambda i: (i, 0)),
            pl.BlockSpec((BLK, D), lambda i: (i, 0)),
            pl.BlockSpec((1, D), lambda i: (0, 0)),
            pl.BlockSpec((1, D), lambda i: (0, 0)),
            pl.BlockSpec((D, D), lambda i: (0, 0)),
            pl.BlockSpec((1, D), lambda i: (0, 0)),
            pl.BlockSpec((D, D), lambda i: (0, 0)),
            pl.BlockSpec((1, D), lambda i: (0, 0)),
            pl.BlockSpec((D, D), lambda i: (0, 0)),
            pl.BlockSpec((1, D), lambda i: (0, 0)),
            pl.BlockSpec((H, D), lambda i: (0, 0)),
        ],
        out_specs=[pl.BlockSpec((BLK, D), lambda i: (i, 0))] * 3,
        out_shape=[jax.ShapeDtypeStruct((NPAD, D), jnp.float32)] * 3,
    )(*args)


EBLK = 1000


def _he_body(ef_ref, we_ref, be_ref, out_ref):
    out_ref[...] = jnp.dot(ef_ref[...], we_ref[...],
                           preferred_element_type=jnp.float32) + be_ref[...]


def _he_proj(edge_features, gp):
    return pl.pallas_call(
        _he_body,
        grid=(E // EBLK,),
        in_specs=[
            pl.BlockSpec((EBLK, DE), lambda i: (i, 0)),
            pl.BlockSpec((DE, D), lambda i: (0, 0)),
            pl.BlockSpec((1, D), lambda i: (0, 0)),
        ],
        out_specs=pl.BlockSpec((EBLK, D), lambda i: (i, 0)),
        out_shape=jax.ShapeDtypeStruct((E, D), jnp.float32),
    )(edge_features, gp['We'], gp['be'].reshape(1, D))


def _pre0_body(x_ref, lns_ref, lnb_ref, wskip_ref, bskip_ref,
               wl_ref, bl_ref, wr_ref, br_ref, skip_ref, hs_ref, hr_ref):
    x = x_ref[...]
    m = x.mean(-1, keepdims=True)
    v = jnp.mean((x - m) * (x - m), axis=-1, keepdims=True)
    x = (x - m) * jax.lax.rsqrt(v + 1e-6) * lns_ref[...] + lnb_ref[...]
    skip_ref[...] = jnp.dot(x, wskip_ref[...], preferred_element_type=jnp.float32) + bskip_ref[...]
    hs_ref[...] = jnp.dot(x, wl_ref[...], preferred_element_type=jnp.float32) + bl_ref[...]
    hr_ref[...] = jnp.dot(x, wr_ref[...], preferred_element_type=jnp.float32) + br_ref[...]


def _pre0(x_pad, lp):
    gp = lp['gat']
    return pl.pallas_call(
        _pre0_body,
        grid=(NPAD // BLK,),
        in_specs=[
            pl.BlockSpec((BLK, D), lambda i: (i, 0)),
            pl.BlockSpec((1, D), lambda i: (0, 0)),
            pl.BlockSpec((1, D), lambda i: (0, 0)),
            pl.BlockSpec((D, D), lambda i: (0, 0)),
            pl.BlockSpec((1, D), lambda i: (0, 0)),
            pl.BlockSpec((D, D), lambda i: (0, 0)),
            pl.BlockSpec((1, D), lambda i: (0, 0)),
            pl.BlockSpec((D, D), lambda i: (0, 0)),
            pl.BlockSpec((1, D), lambda i: (0, 0)),
        ],
        out_specs=[pl.BlockSpec((BLK, D), lambda i: (i, 0))] * 3,
        out_shape=[jax.ShapeDtypeStruct((NPAD, D), jnp.float32)] * 3,
    )(x_pad, lp['ln_s'].reshape(1, D), lp['ln_b'].reshape(1, D),
      lp['skip_W'], lp['skip_b'].reshape(1, D),
      gp['Wl'], gp['bl'].reshape(1, D), gp['Wr'], gp['br'].reshape(1, D))


def _gat_fin_body(num0_ref, num1_ref, den0_ref, den1_ref, skipin_ref,
                  segT_ref, out_ref):
    num = num0_ref[...] + num1_ref[...]
    den = jnp.dot(den0_ref[...] + den1_ref[...], segT_ref[...],
                  preferred_element_type=jnp.float32)
    out_ref[...] = jnp.maximum(num / (den + 1e-9) + skipin_ref[...], 0.0)


def _gat_fin(num, den_t, skipin_pad):
    den0 = den_t[0].reshape(NTAB, H)
    den1 = den_t[1].reshape(NTAB, H)
    segT = jnp.asarray(_SEG).T
    return pl.pallas_call(
        _gat_fin_body,
        grid=(NPAD // BLK,),
        in_specs=[
            pl.BlockSpec((BLK, D), lambda i: (i, 0)),
            pl.BlockSpec((BLK, D), lambda i: (i, 0)),
            pl.BlockSpec((BLK, H), lambda i: (i, 0)),
            pl.BlockSpec((BLK, H), lambda i: (i, 0)),
            pl.BlockSpec((BLK, D), lambda i: (i, 0)),
            pl.BlockSpec((H, D), lambda i: (0, 0)),
        ],
        out_specs=pl.BlockSpec((BLK, D), lambda i: (i, 0)),
        out_shape=jax.ShapeDtypeStruct((NPAD, D), jnp.float32),
    )(num[0], num[1], den0, den1, skipin_pad, segT)


def kernel(node_features, node_mask, edge_features, global_features, edge_list,
           edge_mask, params):
    p = params
    senders = edge_list[:, 0]
    receivers = edge_list[:, 1]
    n = node_features.shape[0]
    nf = jnp.concatenate([node_features, jnp.repeat(global_features, n, axis=0)],
                         axis=-1)
    nf = jnp.concatenate([nf, jnp.zeros((1, nf.shape[-1]), jnp.float32)], axis=0)
    nm_pad = jnp.concatenate([node_mask, jnp.zeros((NPAD - N,), jnp.float32)])
    snd_sc = jnp.where(edge_mask, senders, NTAB - 1).astype(jnp.int32)
    rcv_sc = jnp.where(edge_mask, receivers, NTAB - 1).astype(jnp.int32)
    g = jnp.tile(p['global'], (1, 1))

    nf_pad = jnp.pad(nf, ((0, NPAD - NP1), (0, 0)))
    nodes = _encoder(nf_pad, p)

    g = _attention_pallas(g, nodes, nm_pad, p['attn1'])
    mix_bias = (g @ p['mix_W'][D:] + p['mix_b']).reshape(1, D)
    nodes = _mm(nodes, p['mix_W'][:D], mix_bias, act='relu')

    skip, hs, hr = _pre0(nodes, p['layers'][0])
    for li, lp in enumerate(p['layers']):
        gp = lp['gat']
        he = _he_proj(edge_features, gp)
        att_flat = gp['att'].reshape(1, D)
        hs_tab = jnp.concatenate(
            [hs, att_flat, jnp.zeros((7, D), jnp.float32)], axis=0)
        num, den_t = _sc_edges(hs_tab, hr, he, snd_sc, rcv_sc)
        if li + 1 < L:
            skip, hs, hr = _layer_pre(num, den_t, skip, p['layers'][li + 1])
        else:
            nodes = _gat_fin(num, den_t, skip)
    g = _attention_pallas(g, nodes, nm_pad, p['attn2'])
    g = jax.nn.relu(_ln(g, p['final_ln_s'], p['final_ln_b']))
    return g.reshape(-1)


# speedup vs baseline: 1.6954x; 1.0458x over previous
import functools

import jax
import jax.numpy as jnp
import numpy as np
from jax import lax
from jax.experimental import pallas as pl
from jax.experimental.pallas import tpu as pltpu
from jax.experimental.pallas import tpu_sc as plsc

N = 10000
E = 320000
DFEAT = 128
DG = 32
DE = 16
D = 128
H = 8
DH = D // H
L = 3

NP1 = N + 1
BLK = 128
NPAD = ((NP1 + BLK - 1) // BLK) * BLK

NC = 2
NS = 16
NW = NC * NS
NTAB = 10112
RPT = NTAB // NS
EPW = E // NW
K = 80
DNR = NTAB // 16


def _ln(x, s, b):
    m = x.mean(-1, keepdims=True)
    v = x.var(-1, keepdims=True)
    return (x - m) / jnp.sqrt(v + 1e-6) * s + b


def _encoder_body(nf_ref, w1_ref, b1_ref, lns_ref, lnb_ref, w2_ref, b2_ref,
                  out_ref):
    x = nf_ref[...]
    h = jnp.dot(x, w1_ref[...], preferred_element_type=jnp.float32) + b1_ref[...]
    m = h.mean(-1, keepdims=True)
    v = jnp.mean((h - m) * (h - m), axis=-1, keepdims=True)
    h = (h - m) * jax.lax.rsqrt(v + 1e-6) * lns_ref[...] + lnb_ref[...]
    h = jnp.maximum(h, 0.0)
    out_ref[...] = jnp.dot(h, w2_ref[...], preferred_element_type=jnp.float32) + b2_ref[...]


def _encoder(nf_pad, p):
    grid = NPAD // BLK
    return pl.pallas_call(
        _encoder_body,
        grid=(grid,),
        in_specs=[
            pl.BlockSpec((BLK, DFEAT + DG), lambda i: (i, 0)),
            pl.BlockSpec((DFEAT + DG, D), lambda i: (0, 0)),
            pl.BlockSpec((D,), lambda i: (0,)),
            pl.BlockSpec((D,), lambda i: (0,)),
            pl.BlockSpec((D,), lambda i: (0,)),
            pl.BlockSpec((D, D), lambda i: (0, 0)),
            pl.BlockSpec((D,), lambda i: (0,)),
        ],
        out_specs=pl.BlockSpec((BLK, D), lambda i: (i, 0)),
        out_shape=jax.ShapeDtypeStruct((NPAD, D), jnp.float32),
    )(nf_pad, p['enc_W1'], p['enc_b1'], p['enc_ln_s'], p['enc_ln_b'],
      p['enc_W2'], p['enc_b2'])


def _sc_edge_body(hs_tab, hr_tab, he_hbm, snd, rcv,
                  out_num, out_den,
                  acc, den_acc, att_v, idx_s, idx_r, idx_r2,
                  hs_rows, hr_rows, he_rows, exbuf, staged_den, zbuf,
                  sem_s, sem_r, sem_e):
    cid = lax.axis_index("c")
    sid = lax.axis_index("s")
    wid = cid * NS + sid
    zero16 = jnp.zeros((16,), jnp.float32)
    iota16 = lax.iota(jnp.int32, 16)

    def zrow(r, carry):
        for c in range(8):
            zbuf[r, pl.ds(c * 16, 16)] = zero16
        return carry
    lax.fori_loop(0, 8, zrow, 0)

    def zacc(t, carry):
        pltpu.sync_copy(zbuf, acc.at[pl.ds(sid * RPT + t * 8, 8)])
        return carry
    lax.fori_loop(0, RPT // 8, zacc, 0)
    dstart = jnp.minimum(sid * 40, DNR - 40)
    for t in range(5):
        pltpu.sync_copy(zbuf, den_acc.at[pl.ds(dstart + t * 8, 8)])

    def zsd(r, carry):
        for c in range(8):
            staged_den[r, pl.ds(c * 16, 16)] = zero16
        return carry
    lax.fori_loop(0, K, zsd, 0)
    pltpu.sync_copy(hs_tab.at[pl.ds(NTAB, 1)], att_v)
    plsc.subcore_barrier()

    ebase = wid * EPW

    def batch(b, carry):
        off = ebase + b * K
        pltpu.sync_copy(snd.at[pl.ds(off, K)], idx_s)
        pltpu.sync_copy(rcv.at[pl.ds(off, K)], idx_r)
        cs = pltpu.async_copy(hs_tab.at[idx_s], hs_rows, sem_s)
        cr = pltpu.async_copy(hr_tab.at[idx_r], hr_rows, sem_r)
        ce = pltpu.async_copy(he_hbm.at[pl.ds(off, K)], he_rows, sem_e)
        cs.wait()
        cr.wait()
        ce.wait()

        def p1(k, c1):
            for j in range(H):
                sl = pl.ds(j * 16, 16)
                mv = hs_rows[k, sl] + hr_rows[k, sl] + he_rows[k, sl]
                he_rows[k, sl] = jnp.maximum(mv, mv * 0.2)
            return c1
        lax.fori_loop(0, K, p1, 0)

        def p2(g, c2):
            rows = g * 16 + iota16
            rvec = idx_r[pl.ds(g * 16, 16)]
            idx_r2[pl.ds(g * 16, 16)] = lax.shift_right_logical(rvec, 4)
            posv = (rvec & 15) * 8
            for h in range(H):
                attv = att_v[0, pl.ds(h * 16, 16)]
                lg = jnp.zeros((16,), jnp.float32)
                for dd in range(16):
                    col = plsc.load_gather(
                        he_rows, [rows, jnp.full((16,), h * 16 + dd, jnp.int32)])
                    lg = lg + col * attv[dd]
                exh = jnp.exp(lg)
                plsc.store_scatter(exbuf, [g * 256 + iota16 * 16 + h], exh)
                plsc.store_scatter(staged_den, [rows, posv + h], exh)
            return c2
        lax.fori_loop(0, K // 16, p2, 0)

        def p3(k, c3):
            exrow = exbuf[pl.ds(k * 16, 16)]
            for j in range(H):
                sl = pl.ds(j * 16, 16)
                hr_rows[k, sl] = hs_rows[k, sl] * exrow[j]
            return c3
        lax.fori_loop(0, K, p3, 0)

        a1 = pltpu.async_copy(hr_rows, acc.at[idx_r], sem_s, add=True)
        a2 = pltpu.async_copy(staged_den, den_acc.at[idx_r2], sem_r, add=True)
        a2.wait()

        def pc(g, c4):
            rows = g * 16 + iota16
            posv = (idx_r[pl.ds(g * 16, 16)] & 15) * 8
            for h in range(H):
                plsc.store_scatter(staged_den, [rows, posv + h], zero16)
            return c4
        lax.fori_loop(0, K // 16, pc, 0)
        a1.wait()
        return carry
    lax.fori_loop(0, EPW // K, batch, 0)

    plsc.subcore_barrier()
    pltpu.sync_copy(acc.at[pl.ds(sid * RPT, RPT)],
                    out_num.at[cid, pl.ds(sid * RPT, RPT)])
    dstart2 = jnp.minimum(sid * 40, DNR - 40)
    pltpu.sync_copy(den_acc.at[pl.ds(dstart2, 40)],
                    out_den.at[cid, pl.ds(dstart2, 40)])


_sc_edges_built = None


def _build_sc_edges():
    return pl.kernel(
        _sc_edge_body,
        out_type=(jax.ShapeDtypeStruct((NC, NTAB, D), jnp.float32),
                  jax.ShapeDtypeStruct((NC, DNR, D), jnp.float32)),
        mesh=plsc.VectorSubcoreMesh(core_axis_name="c", subcore_axis_name="s",
                                    num_cores=NC, num_subcores=NS),
        scratch_types=[
            pltpu.VMEM_SHARED((NTAB, D), jnp.float32),
            pltpu.VMEM_SHARED((DNR, D), jnp.float32),
            pltpu.VMEM((1, D), jnp.float32),
            pltpu.VMEM((K,), jnp.int32),
            pltpu.VMEM((K,), jnp.int32),
            pltpu.VMEM((K,), jnp.int32),
            pltpu.VMEM((K, D), jnp.float32),
            pltpu.VMEM((K, D), jnp.float32),
            pltpu.VMEM((K, D), jnp.float32),
            pltpu.VMEM((K * 16,), jnp.float32),
            pltpu.VMEM((K, D), jnp.float32),
            pltpu.VMEM((8, D), jnp.float32),
            pltpu.SemaphoreType.DMA,
            pltpu.SemaphoreType.DMA,
            pltpu.SemaphoreType.DMA,
        ],
        compiler_params=pltpu.CompilerParams(needs_layout_passes=False),
        name="sc_gatv2_edges",
    )


def _sc_edges(*args):
    global _sc_edges_built
    if _sc_edges_built is None:
        _sc_edges_built = _build_sc_edges()
    return _sc_edges_built(*args)


_SEG = np.kron(np.eye(H, dtype=np.float32), np.ones((DH, 1), np.float32))


_P = np.kron(np.eye(H, dtype=np.float32), np.ones((DH, DH), np.float32))


def _attn_body(nodes_ref, nm_ref, wk_ref, bk_ref, wv_ref, bv_ref,
               klns_ref, klnb_ref, q_ref, pmat_ref,
               out_ref, macc_ref, sacc_ref, vacc_ref):
    i = pl.program_id(0)

    @pl.when(i == 0)
    def _init():
        macc_ref[...] = jnp.full((1, D), -1e30, jnp.float32)
        sacc_ref[...] = jnp.zeros((1, D), jnp.float32)
        vacc_ref[...] = jnp.zeros((1, D), jnp.float32)

    x = nodes_ref[...]
    pmat = pmat_ref[...]
    k = jnp.dot(x, wk_ref[...], preferred_element_type=jnp.float32) + bk_ref[...]
    mfull = jnp.dot(k, pmat, preferred_element_type=jnp.float32) * (1.0 / DH)
    kc = k - mfull
    vfull = jnp.dot(kc * kc, pmat, preferred_element_type=jnp.float32) * (1.0 / DH)
    kln = kc * jax.lax.rsqrt(vfull + 1e-6) * klns_ref[...] + klnb_ref[...]
    sfull = jnp.dot(kln * q_ref[...], pmat,
                    preferred_element_type=jnp.float32) * (1.0 / np.sqrt(DH))
    sfull = jnp.where(nm_ref[...] > 0.0, sfull, -1e9)
    bm = jnp.max(sfull, axis=0, keepdims=True)
    m_old = macc_ref[...]
    m_new = jnp.maximum(m_old, bm)
    corr = jnp.exp(m_old - m_new)
    w = jnp.exp(sfull - m_new)
    v = jnp.dot(x, wv_ref[...], preferred_element_type=jnp.float32) + bv_ref[...]
    macc_ref[...] = m_new
    sacc_ref[...] = sacc_ref[...] * corr + jnp.sum(w, axis=0, keepdims=True)
    vacc_ref[...] = vacc_ref[...] * corr + jnp.sum(w * v, axis=0, keepdims=True)

    @pl.when(i == pl.num_programs(0) - 1)
    def _fin():
        out_ref[...] = vacc_ref[...] / sacc_ref[...]


def _attention_pallas(g, nodes_pad, nm2d, ap):
    q = (g @ ap['Wq'] + ap['bq']).reshape(1, H, DH)
    q = _ln(q, ap['qln_s'], ap['qln_b']).reshape(1, D)
    klns = jnp.tile(ap['kln_s'], H).reshape(1, D)
    klnb = jnp.tile(ap['kln_b'], H).reshape(1, D)
    ABLK = 632
    out = pl.pallas_call(
        _attn_body,
        grid=(NPAD // ABLK,),
        in_specs=[
            pl.BlockSpec((ABLK, D), lambda i: (i, 0)),
            pl.BlockSpec((ABLK, D), lambda i: (i, 0)),
            pl.BlockSpec((D, D), lambda i: (0, 0)),
            pl.BlockSpec((D,), lambda i: (0,)),
            pl.BlockSpec((D, D), lambda i: (0, 0)),
            pl.BlockSpec((D,), lambda i: (0,)),
            pl.BlockSpec((1, D), lambda i: (0, 0)),
            pl.BlockSpec((1, D), lambda i: (0, 0)),
            pl.BlockSpec((1, D), lambda i: (0, 0)),
            pl.BlockSpec((D, D), lambda i: (0, 0)),
        ],
        out_specs=pl.BlockSpec((1, D), lambda i: (0, 0)),
        out_shape=jax.ShapeDtypeStruct((1, D), jnp.float32),
        scratch_shapes=[
            pltpu.VMEM((1, D), jnp.float32),
            pltpu.VMEM((1, D), jnp.float32),
            pltpu.VMEM((1, D), jnp.float32),
        ],
    )(nodes_pad, nm2d,
      ap['Wk'], ap['bk'], ap['Wv'], ap['bv'],
      klns, klnb, q, jnp.asarray(_P))
    return g + out @ ap['Wo'] + ap['bo']


def _mm_ln_body(x_ref, w_ref, b_ref, out_ref, *, act):
    h = jnp.dot(x_ref[...], w_ref[...], preferred_element_type=jnp.float32) + b_ref[...]
    if act == 'relu':
        h = jnp.maximum(h, 0.0)
    out_ref[...] = h


def _mm(x_pad, w, b, act='none'):
    return pl.pallas_call(
        functools.partial(_mm_ln_body, act=act),
        grid=(NPAD // BLK,),
        in_specs=[
            pl.BlockSpec((BLK, D), lambda i: (i, 0)),
            pl.BlockSpec((D, D), lambda i: (0, 0)),
            pl.BlockSpec((1, D), lambda i: (0, 0)),
        ],
        out_specs=pl.BlockSpec((BLK, D), lambda i: (i, 0)),
        out_shape=jax.ShapeDtypeStruct((NPAD, D), jnp.float32),
    )(x_pad, w, b.reshape(1, D))


def _layer_pre_body(num0_ref, num1_ref, den0_ref, den1_ref, skipin_ref,
                    lns_ref, lnb_ref, wskip_ref, bskip_ref,
                    wl_ref, bl_ref, wr_ref, br_ref, segT_ref,
                    skip_ref, hs_ref, hr_ref):
    num = num0_ref[...] + num1_ref[...]
    den = jnp.dot(den0_ref[...] + den1_ref[...], segT_ref[...],
                  preferred_element_type=jnp.float32)
    gat = num / (den + 1e-9)
    x = jnp.maximum(gat + skipin_ref[...], 0.0)
    m = x.mean(-1, keepdims=True)
    v = jnp.mean((x - m) * (x - m), axis=-1, keepdims=True)
    x = (x - m) * jax.lax.rsqrt(v + 1e-6) * lns_ref[...] + lnb_ref[...]
    skip_ref[...] = jnp.dot(x, wskip_ref[...], preferred_element_type=jnp.float32) + bskip_ref[...]
    hs_ref[...] = jnp.dot(x, wl_ref[...], preferred_element_type=jnp.float32) + bl_ref[...]
    hr_ref[...] = jnp.dot(x, wr_ref[...], preferred_element_type=jnp.float32) + br_ref[...]


def _layer_pre(num, den_t, skipin_pad, lp):
    gp = lp['gat']
    den0 = den_t[0].reshape(NTAB, H)
    den1 = den_t[1].reshape(NTAB, H)
    segT = jnp.asarray(_SEG).T
    args = (num[0], num[1], den0, den1, skipin_pad,
            lp['ln_s'].reshape(1, D), lp['ln_b'].reshape(1, D),
            lp['skip_W'], lp['skip_b'].reshape(1, D),
            gp['Wl'], gp['bl'].reshape(1, D),
            gp['Wr'], gp['br'].reshape(1, D), segT)
    return pl.pallas_call(
        _layer_pre_body,
        grid=(NPAD // BLK,),
        in_specs=[
            pl.BlockSpec((BLK, D), lambda i: (i, 0)),
            pl.BlockSpec((BLK, D), lambda i: (i, 0)),
            pl.BlockSpec((BLK, H), lambda i: (i, 0)),
            pl.BlockSpec((BLK, H), lambda i: (i, 0)),
            pl.BlockSpec((BLK, D), lambda i: (i, 0)),
            pl.BlockSpec((1, D), lambda i: (0, 0)),
            pl.BlockSpec((1, D), lambda i: (0, 0)),
            pl.BlockSpec((D, D), lambda i: (0, 0)),
            pl.BlockSpec((1, D), lambda i: (0, 0)),
            pl.BlockSpec((D, D), lambda i: (0, 0)),
            pl.BlockSpec((1, D), lambda i: (0, 0)),
            pl.BlockSpec((D, D), lambda i: (0, 0)),
            pl.BlockSpec((1, D), lambda i: (0, 0)),
            pl.BlockSpec((H, D), lambda i: (0, 0)),
        ],
        out_specs=[pl.BlockSpec((BLK, D), lambda i: (i, 0))] * 3,
        out_shape=[jax.ShapeDtypeStruct((NPAD, D), jnp.float32)] * 3,
    )(*args)


EBLK = 4000


def _he_body(ef_ref, we_ref, be_ref, out_ref):
    out_ref[...] = jnp.dot(ef_ref[...], we_ref[...],
                           preferred_element_type=jnp.float32) + be_ref[...]


def _he_proj(edge_features, gp):
    return pl.pallas_call(
        _he_body,
        grid=(E // EBLK,),
        in_specs=[
            pl.BlockSpec((EBLK, DE), lambda i: (i, 0)),
            pl.BlockSpec((DE, D), lambda i: (0, 0)),
            pl.BlockSpec((1, D), lambda i: (0, 0)),
        ],
        out_specs=pl.BlockSpec((EBLK, D), lambda i: (i, 0)),
        out_shape=jax.ShapeDtypeStruct((E, D), jnp.float32),
    )(edge_features, gp['We'], gp['be'].reshape(1, D))


def _pre0_body(x_ref, lns_ref, lnb_ref, wskip_ref, bskip_ref,
               wl_ref, bl_ref, wr_ref, br_ref, skip_ref, hs_ref, hr_ref):
    x = x_ref[...]
    m = x.mean(-1, keepdims=True)
    v = jnp.mean((x - m) * (x - m), axis=-1, keepdims=True)
    x = (x - m) * jax.lax.rsqrt(v + 1e-6) * lns_ref[...] + lnb_ref[...]
    skip_ref[...] = jnp.dot(x, wskip_ref[...], preferred_element_type=jnp.float32) + bskip_ref[...]
    hs_ref[...] = jnp.dot(x, wl_ref[...], preferred_element_type=jnp.float32) + bl_ref[...]
    hr_ref[...] = jnp.dot(x, wr_ref[...], preferred_element_type=jnp.float32) + br_ref[...]


def _pre0(x_pad, lp):
    gp = lp['gat']
    return pl.pallas_call(
        _pre0_body,
        grid=(NPAD // BLK,),
        in_specs=[
            pl.BlockSpec((BLK, D), lambda i: (i, 0)),
            pl.BlockSpec((1, D), lambda i: (0, 0)),
            pl.BlockSpec((1, D), lambda i: (0, 0)),
            pl.BlockSpec((D, D), lambda i: (0, 0)),
            pl.BlockSpec((1, D), lambda i: (0, 0)),
            pl.BlockSpec((D, D), lambda i: (0, 0)),
            pl.BlockSpec((1, D), lambda i: (0, 0)),
            pl.BlockSpec((D, D), lambda i: (0, 0)),
            pl.BlockSpec((1, D), lambda i: (0, 0)),
        ],
        out_specs=[pl.BlockSpec((BLK, D), lambda i: (i, 0))] * 3,
        out_shape=[jax.ShapeDtypeStruct((NPAD, D), jnp.float32)] * 3,
    )(x_pad, lp['ln_s'].reshape(1, D), lp['ln_b'].reshape(1, D),
      lp['skip_W'], lp['skip_b'].reshape(1, D),
      gp['Wl'], gp['bl'].reshape(1, D), gp['Wr'], gp['br'].reshape(1, D))


def _gat_fin_body(num0_ref, num1_ref, den0_ref, den1_ref, skipin_ref,
                  segT_ref, out_ref):
    num = num0_ref[...] + num1_ref[...]
    den = jnp.dot(den0_ref[...] + den1_ref[...], segT_ref[...],
                  preferred_element_type=jnp.float32)
    out_ref[...] = jnp.maximum(num / (den + 1e-9) + skipin_ref[...], 0.0)


def _gat_fin(num, den_t, skipin_pad):
    den0 = den_t[0].reshape(NTAB, H)
    den1 = den_t[1].reshape(NTAB, H)
    segT = jnp.asarray(_SEG).T
    return pl.pallas_call(
        _gat_fin_body,
        grid=(NPAD // BLK,),
        in_specs=[
            pl.BlockSpec((BLK, D), lambda i: (i, 0)),
            pl.BlockSpec((BLK, D), lambda i: (i, 0)),
            pl.BlockSpec((BLK, H), lambda i: (i, 0)),
            pl.BlockSpec((BLK, H), lambda i: (i, 0)),
            pl.BlockSpec((BLK, D), lambda i: (i, 0)),
            pl.BlockSpec((H, D), lambda i: (0, 0)),
        ],
        out_specs=pl.BlockSpec((BLK, D), lambda i: (i, 0)),
        out_shape=jax.ShapeDtypeStruct((NPAD, D), jnp.float32),
    )(num[0], num[1], den0, den1, skipin_pad, segT)


def kernel(node_features, node_mask, edge_features, global_features, edge_list,
           edge_mask, params):
    p = params
    senders = edge_list[:, 0]
    receivers = edge_list[:, 1]
    n = node_features.shape[0]
    nf = jnp.concatenate([node_features, jnp.repeat(global_features, n, axis=0)],
                         axis=-1)
    nf = jnp.concatenate([nf, jnp.zeros((1, nf.shape[-1]), jnp.float32)], axis=0)
    nm_pad = jnp.concatenate([node_mask, jnp.zeros((NPAD - N,), jnp.float32)])
    snd_sc = jnp.where(edge_mask, senders, NTAB - 1).astype(jnp.int32)
    rcv_sc = jnp.where(edge_mask, receivers, NTAB - 1).astype(jnp.int32)
    g = jnp.tile(p['global'], (1, 1))

    nf_pad = jnp.pad(nf, ((0, NPAD - NP1), (0, 0)))
    nodes = _encoder(nf_pad, p)

    nm2d = jnp.repeat(nm_pad[:, None], D, axis=1)
    g = _attention_pallas(g, nodes, nm2d, p['attn1'])
    mix_bias = (g @ p['mix_W'][D:] + p['mix_b']).reshape(1, D)
    nodes = _mm(nodes, p['mix_W'][:D], mix_bias, act='relu')

    skip, hs, hr = _pre0(nodes, p['layers'][0])
    for li, lp in enumerate(p['layers']):
        gp = lp['gat']
        he = _he_proj(edge_features, gp)
        att_flat = gp['att'].reshape(1, D)
        hs_tab = jnp.concatenate(
            [hs, att_flat, jnp.zeros((7, D), jnp.float32)], axis=0)
        num, den_t = _sc_edges(hs_tab, hr, he, snd_sc, rcv_sc)
        if li + 1 < L:
            skip, hs, hr = _layer_pre(num, den_t, skip, p['layers'][li + 1])
        else:
            nodes = _gat_fin(num, den_t, skip)
    g = _attention_pallas(g, nodes, nm2d, p['attn2'])
    g = jax.nn.relu(_ln(g, p['final_ln_s'], p['final_ln_b']))
    return g.reshape(-1)
